# baseline (speedup 1.0000x reference)
import functools

import jax
import jax.numpy as jnp
from jax.experimental import pallas as pl

HID = 64
RELS = ['CC', 'CH', 'HH', 'CO', 'HO', 'OO']


def _mlp2_body(x_ref, w1_ref, b1_ref, w2_ref, b2_ref, o_ref, *, relu_out):
    h = jnp.maximum(x_ref[...] @ w1_ref[...] + b1_ref[...], 0.0)
    o = h @ w2_ref[...] + b2_ref[...]
    if relu_out:
        o = jnp.maximum(o, 0.0)
    o_ref[...] = o


def _mlp2(x, w1, b1, w2, b2, blk, relu_out=False):
    n, din = x.shape
    return pl.pallas_call(
        functools.partial(_mlp2_body, relu_out=relu_out),
        grid=(n // blk,),
        in_specs=[
            pl.BlockSpec((blk, din), lambda i: (i, 0)),
            pl.BlockSpec((din, HID), lambda i: (0, 0)),
            pl.BlockSpec((1, HID), lambda i: (0, 0)),
            pl.BlockSpec((HID, HID), lambda i: (0, 0)),
            pl.BlockSpec((1, HID), lambda i: (0, 0)),
        ],
        out_specs=pl.BlockSpec((blk, HID), lambda i: (i, 0)),
        out_shape=jax.ShapeDtypeStruct((n, HID), jnp.float32),
    )(x, w1, b1.reshape(1, HID), w2, b2.reshape(1, HID))


def _edge_t_body(ea_ref, w1_ref, b1_ref, wa_ref, ba_ref, wb_ref, bb_ref,
                 ta_ref, tb_ref):
    h = jnp.maximum(ea_ref[...] @ w1_ref[...] + b1_ref[...], 0.0)
    ta_ref[...] = h @ wa_ref[...] + ba_ref[...]
    tb_ref[...] = h @ wb_ref[...] + bb_ref[...]


def _edge_t(ea, ew1, eb1, wa, ba, wb, bb, blk):
    n, din = ea.shape
    out = jax.ShapeDtypeStruct((n, HID), jnp.float32)
    return pl.pallas_call(
        _edge_t_body,
        grid=(n // blk,),
        in_specs=[
            pl.BlockSpec((blk, din), lambda i: (i, 0)),
            pl.BlockSpec((din, HID), lambda i: (0, 0)),
            pl.BlockSpec((1, HID), lambda i: (0, 0)),
            pl.BlockSpec((HID, HID), lambda i: (0, 0)),
            pl.BlockSpec((1, HID), lambda i: (0, 0)),
            pl.BlockSpec((HID, HID), lambda i: (0, 0)),
            pl.BlockSpec((1, HID), lambda i: (0, 0)),
        ],
        out_specs=[pl.BlockSpec((blk, HID), lambda i: (i, 0))] * 2,
        out_shape=[out, out],
    )(ea, ew1, eb1.reshape(1, HID), wa, ba.reshape(1, HID), wb,
      bb.reshape(1, HID))


def _node_update_body(*refs, k):
    x_ref = refs[0]
    agg_refs = refs[1:1 + k]
    s_ref, w1_ref, b1_ref, w2_ref, b2_ref, o_ref = refs[1 + k:]
    acc = None
    for i in range(k):
        z = x_ref[...] * s_ref[0, i] + agg_refs[i][...]
        h = jnp.maximum(z @ w1_ref[...] + b1_ref[...], 0.0)
        y = h @ w2_ref[...] + b2_ref[...]
        acc = y if acc is None else acc + y
    o_ref[...] = jnp.maximum(acc, 0.0)


def _node_update(x, aggs, scales, nw1, nb1, nw2, nb2, blk):
    n = x.shape[0]
    k = len(aggs)
    in_specs = [pl.BlockSpec((blk, HID), lambda i: (i, 0))]
    in_specs += [pl.BlockSpec((blk, HID), lambda i: (i, 0))] * k
    in_specs += [
        pl.BlockSpec((1, k), lambda i: (0, 0)),
        pl.BlockSpec((HID, HID), lambda i: (0, 0)),
        pl.BlockSpec((1, HID), lambda i: (0, 0)),
        pl.BlockSpec((HID, HID), lambda i: (0, 0)),
        pl.BlockSpec((1, HID), lambda i: (0, 0)),
    ]
    return pl.pallas_call(
        functools.partial(_node_update_body, k=k),
        grid=(n // blk,),
        in_specs=in_specs,
        out_specs=pl.BlockSpec((blk, HID), lambda i: (i, 0)),
        out_shape=jax.ShapeDtypeStruct((n, HID), jnp.float32),
    )(x, *aggs, scales.reshape(1, k), nw1, nb1.reshape(1, HID), nw2,
      nb2.reshape(1, HID))


def _head_body(x_ref, w_ref, b_ref, o_ref):
    o_ref[...] = x_ref[...] @ w_ref[...] + b_ref[...]


def _head(x, w, b, blk):
    n = x.shape[0]
    return pl.pallas_call(
        _head_body,
        grid=(n // blk,),
        in_specs=[
            pl.BlockSpec((blk, HID), lambda i: (i, 0)),
            pl.BlockSpec((HID, 1), lambda i: (0, 0)),
            pl.BlockSpec((1, 1), lambda i: (0, 0)),
        ],
        out_specs=pl.BlockSpec((blk, 1), lambda i: (i, 0)),
        out_shape=jax.ShapeDtypeStruct((n, 1), jnp.float32),
    )(x, w, b.reshape(1, 1))


def kernel(x_C, x_H, x_O, ei_CC, ea_CC, ei_CH, ea_CH, ei_HH, ea_HH, ei_CO,
           ea_CO, ei_HO, ea_HO, ei_OO, ea_OO, cW1, cb1, cW2, cb2, hW1, hb1,
           hW2, hb2, oW1, ob1, oW2, ob2, eW1, eb1, eW2, eb2, nW1, nb1, nW2,
           nb2, linW, linb, eps, outcW, outcb, outhW, outhb):
    blk = 2000
    ei = {r: e for r, e in zip(RELS, (ei_CC, ei_CH, ei_HH, ei_CO, ei_HO,
                                      ei_OO))}
    ea = {r: e for r, e in zip(RELS, (ea_CC, ea_CH, ea_HH, ea_CO, ea_HO,
                                      ea_OO))}

    wc = jnp.einsum('dk,jkh->jdh', eW2, linW)
    bc = jnp.einsum('k,jkh->jh', eb2, linW) + linb

    x = {
        'C': _mlp2(x_C, cW1, cb1, cW2, cb2, blk),
        'H': _mlp2(x_H, hW1, hb1, hW2, hb2, blk),
        'O': _mlp2(x_O, oW1, ob1, oW2, ob2, blk),
    }
    t = {}
    for i, r in enumerate(RELS):
        t0, t1 = _edge_t(ea[r], eW1, eb1, wc[i], bc[i], wc[i + 6], bc[i + 6],
                         blk)
        t[r] = (t0, t1)

    nsz = {'C': x_C.shape[0], 'H': x_H.shape[0], 'O': x_O.shape[0]}
    for layer in range(2):
        agg = {}
        for i, r in enumerate(RELS):
            src, dst = ei[r][0], ei[r][1]
            msg = jnp.maximum(x[r[0]][src] + t[r][layer], 0.0)
            agg[r] = jax.ops.segment_sum(msg, dst, num_segments=nsz[r[1]])
        s = 1.0 + eps[layer * 6:layer * 6 + 6]
        x = {
            'C': _node_update(x['C'], [agg['CC']], s[0:1], nW1, nb1, nW2,
                              nb2, blk),
            'H': _node_update(x['H'], [agg['CH'], agg['HH']], s[1:3], nW1,
                              nb1, nW2, nb2, blk),
            'O': _node_update(x['O'], [agg['CO'], agg['HO'], agg['OO']],
                              s[3:6], nW1, nb1, nW2, nb2, blk),
        }
    out_c = _head(x['C'], outcW, outcb, blk)
    out_h = _head(x['H'], outhW, outhb, blk)
    return out_c, out_h


# speedup vs baseline: 1.1490x; 1.1490x over previous
import functools

import jax
import jax.numpy as jnp
from jax import lax
from jax.experimental import pallas as pl
from jax.experimental.pallas import tpu as pltpu
from jax.experimental.pallas import tpu_sc as plsc

HID = 64
RELS = ['CC', 'CH', 'HH', 'CO', 'HO', 'OO']
NT = {'C': 50000, 'H': 100000, 'O': 10000}
NP = {'C': 50048, 'H': 100096, 'O': 10048}
ACC_ROWS = 100096
EPAD_UNIT = 8192


def _ceil_to(x, m):
    return (x + m - 1) // m * m


def _mlp2_body(x_ref, w1_ref, b1_ref, w2_ref, b2_ref, o_ref, *, relu_out):
    h = jnp.maximum(x_ref[...] @ w1_ref[...] + b1_ref[...], 0.0)
    o = h @ w2_ref[...] + b2_ref[...]
    if relu_out:
        o = jnp.maximum(o, 0.0)
    o_ref[...] = o


def _mlp2(x, w1, b1, w2, b2, blk, relu_out=False):
    n, din = x.shape
    return pl.pallas_call(
        functools.partial(_mlp2_body, relu_out=relu_out),
        grid=(n // blk,),
        in_specs=[
            pl.BlockSpec((blk, din), lambda i: (i, 0)),
            pl.BlockSpec((din, HID), lambda i: (0, 0)),
            pl.BlockSpec((1, HID), lambda i: (0, 0)),
            pl.BlockSpec((HID, HID), lambda i: (0, 0)),
            pl.BlockSpec((1, HID), lambda i: (0, 0)),
        ],
        out_specs=pl.BlockSpec((blk, HID), lambda i: (i, 0)),
        out_shape=jax.ShapeDtypeStruct((n, HID), jnp.float32),
    )(x, w1, b1.reshape(1, HID), w2, b2.reshape(1, HID))


def _edge_t_body(ea_ref, w1_ref, b1_ref, wa_ref, ba_ref, wb_ref, bb_ref,
                 ta_ref, tb_ref, *, blk, n_real):
    h = jnp.maximum(ea_ref[...] @ w1_ref[...] + b1_ref[...], 0.0)
    ta = h @ wa_ref[...] + ba_ref[...]
    tb = h @ wb_ref[...] + bb_ref[...]
    rows = pl.program_id(0) * blk + lax.broadcasted_iota(jnp.int32, (blk, 1), 0)
    pad = rows >= n_real
    ta_ref[...] = jnp.where(pad, -1e9, ta)
    tb_ref[...] = jnp.where(pad, -1e9, tb)


def _edge_t(ea, ew1, eb1, wa, ba, wb, bb, blk, n_real):
    n, din = ea.shape
    out = jax.ShapeDtypeStruct((n, HID), jnp.float32)
    return pl.pallas_call(
        functools.partial(_edge_t_body, blk=blk, n_real=n_real),
        grid=(n // blk,),
        in_specs=[
            pl.BlockSpec((blk, din), lambda i: (i, 0)),
            pl.BlockSpec((din, HID), lambda i: (0, 0)),
            pl.BlockSpec((1, HID), lambda i: (0, 0)),
            pl.BlockSpec((HID, HID), lambda i: (0, 0)),
            pl.BlockSpec((1, HID), lambda i: (0, 0)),
            pl.BlockSpec((HID, HID), lambda i: (0, 0)),
            pl.BlockSpec((1, HID), lambda i: (0, 0)),
        ],
        out_specs=[pl.BlockSpec((blk, HID), lambda i: (i, 0))] * 2,
        out_shape=[out, out],
    )(ea, ew1, eb1.reshape(1, HID), wa, ba.reshape(1, HID), wb,
      bb.reshape(1, HID))


def _node_update_body(*refs, k):
    x_ref = refs[0]
    agg_refs = refs[1:1 + k]
    s_ref, w1_ref, b1_ref, w2_ref, b2_ref, o_ref = refs[1 + k:]
    acc = None
    for i in range(k):
        z = x_ref[...] * s_ref[0, i] + agg_refs[i][...]
        h = jnp.maximum(z @ w1_ref[...] + b1_ref[...], 0.0)
        y = h @ w2_ref[...] + b2_ref[...]
        acc = y if acc is None else acc + y
    o_ref[...] = jnp.maximum(acc, 0.0)


def _node_update(x, aggs, scales, nw1, nb1, nw2, nb2, blk):
    n = x.shape[0]
    k = len(aggs)
    in_specs = [pl.BlockSpec((blk, HID), lambda i: (i, 0))]
    in_specs += [pl.BlockSpec((blk, HID), lambda i: (i, 0))] * k
    in_specs += [
        pl.BlockSpec((1, k), lambda i: (0, 0)),
        pl.BlockSpec((HID, HID), lambda i: (0, 0)),
        pl.BlockSpec((1, HID), lambda i: (0, 0)),
        pl.BlockSpec((HID, HID), lambda i: (0, 0)),
        pl.BlockSpec((1, HID), lambda i: (0, 0)),
    ]
    return pl.pallas_call(
        functools.partial(_node_update_body, k=k),
        grid=(n // blk,),
        in_specs=in_specs,
        out_specs=pl.BlockSpec((blk, HID), lambda i: (i, 0)),
        out_shape=jax.ShapeDtypeStruct((n, HID), jnp.float32),
    )(x, *aggs, scales.reshape(1, k), nw1, nb1.reshape(1, HID), nw2,
      nb2.reshape(1, HID))


def _head_body(x_ref, w_ref, b_ref, o_ref):
    o_ref[...] = x_ref[...] @ w_ref[...] + b_ref[...]


def _head(x, w, b, blk):
    n = x.shape[0]
    return pl.pallas_call(
        _head_body,
        grid=(n // blk,),
        in_specs=[
            pl.BlockSpec((blk, HID), lambda i: (i, 0)),
            pl.BlockSpec((HID, 1), lambda i: (0, 0)),
            pl.BlockSpec((1, 1), lambda i: (0, 0)),
        ],
        out_specs=pl.BlockSpec((blk, 1), lambda i: (i, 0)),
        out_shape=jax.ShapeDtypeStruct((n, 1), jnp.float32),
    )(x, w, b.reshape(1, 1))


BLK_E = 512


def _sc_layer_kernel(epad):
    mesh = plsc.VectorSubcoreMesh(core_axis_name="c", subcore_axis_name="s")
    f32 = jnp.float32
    i32 = jnp.int32
    out_type = [
        jax.ShapeDtypeStruct((4, NP['C'], 16), f32),
        jax.ShapeDtypeStruct((4, NP['H'], 16), f32),
        jax.ShapeDtypeStruct((4, NP['H'], 16), f32),
        jax.ShapeDtypeStruct((4, NP['O'], 16), f32),
        jax.ShapeDtypeStruct((4, NP['O'], 16), f32),
        jax.ShapeDtypeStruct((4, NP['O'], 16), f32),
    ]
    scratch_types = [
        pltpu.VMEM_SHARED((ACC_ROWS, 16), f32),
        pltpu.VMEM((BLK_E, 16), f32),
        pltpu.VMEM((BLK_E, 16), f32),
        pltpu.VMEM((BLK_E,), i32),
        pltpu.VMEM((BLK_E,), i32),
        pltpu.VMEM((4, 128), i32),
        pltpu.VMEM((4, 128), i32),
        pltpu.VMEM((4, 128), i32),
        pltpu.VMEM((391, 16), f32),
        pltpu.SemaphoreType.DMA,
    ]

    @functools.partial(pl.kernel, out_type=out_type, mesh=mesh,
                       scratch_types=scratch_types, name="gine_sc_layer",
                       compiler_params=pltpu.CompilerParams(
                           use_tc_tiling_on_sc=False))
    def k(xC, xH, xO,
          sCC, dCC, tCC, sCH, dCH, tCH, sHH, dHH, tHH,
          sCO, dCO, tCO, sHO, dHO, tHO, sOO, dOO, tOO,
          aggCC, aggCH, aggHH, aggCO, aggHO, aggOO,
          acc, tb, msg, sraw, draw, gidx, didx, tidx, zb, sem):
        c = lax.axis_index("c")
        s = lax.axis_index("s")
        half = s // 8
        rank8 = s % 8
        tio = lax.iota(i32, 16) * 4

        def zfill(i, _):
            zb[i] = jnp.zeros((16,), f32)
            return 0
        lax.fori_loop(0, 391, zfill, 0)

        def pass_scan(src_h, dst_h, t_h, x_h, q, accbase, rank, ntiles, ep):
            cnt = ep // ntiles
            nblk = cnt // BLK_E
            base = rank * cnt

            def blk_body(j, _):
                eoff = base + j * BLK_E
                pltpu.sync_copy(src_h.at[pl.ds(eoff, BLK_E)], sraw)
                pltpu.sync_copy(dst_h.at[pl.ds(eoff, BLK_E)], draw)
                for u in range(4):
                    for l in range(8):
                        o = u * 128 + l * 16
                        sv = sraw[pl.ds(o, 16)]
                        gidx[u, pl.ds(l * 16, 16)] = (sv << 2) + q
                        dv = draw[pl.ds(o, 16)]
                        didx[u, pl.ds(l * 16, 16)] = dv + accbase
                        tidx[u, pl.ds(l * 16, 16)] = ((eoff + o) << 2) + q + tio
                cps = []
                for u in range(4):
                    cps.append(pltpu.async_copy(
                        t_h.at[tidx.at[u]], tb.at[pl.ds(u * 128, 128)], sem))
                    cps.append(pltpu.async_copy(
                        x_h.at[gidx.at[u]], msg.at[pl.ds(u * 128, 128)], sem))
                for cp in cps:
                    cp.wait()

                def rb(i, _):
                    b2 = i * 8
                    for k2 in range(8):
                        r2 = b2 + k2
                        msg[r2] = jnp.maximum(msg[r2] + tb[r2], 0.0)
                    return 0
                lax.fori_loop(0, BLK_E // 8, rb, 0)

                for u in range(4):
                    pltpu.sync_copy(msg.at[pl.ds(u * 128, 128)],
                                    acc.at[didx.at[u]], add=True)
                return 0

            lax.fori_loop(0, nblk, blk_body, 0)

        def drain(agg, q, a0, r0, nr):
            pltpu.sync_copy(acc.at[pl.ds(a0, nr)], agg.at[q, pl.ds(r0, nr)])

        def zero_rows():
            z0 = s * 6256
            def zbody(i, _):
                pltpu.sync_copy(zb, acc.at[pl.ds(z0 + i * 391, 391)])
                return 0
            lax.fori_loop(0, 16, zbody, 0)

        qC = 2 * c + half
        zero_rows()
        plsc.subcore_barrier()
        pass_scan(sCC, dCC, tCC, xC, qC, half * NP['C'], rank8, 8, epad['CC'])
        plsc.subcore_barrier()
        drain(aggCC, qC, half * NP['C'] + rank8 * 6256, rank8 * 6256, 6256)
        plsc.subcore_barrier()

        for (sh, dh, th, xs, agg, ep) in (
                (sCH, dCH, tCH, xC, aggCH, epad['CH']),
                (sHH, dHH, tHH, xH, aggHH, epad['HH'])):
            for k2 in range(2):
                qH = 2 * c + k2
                zero_rows()
                plsc.subcore_barrier()
                pass_scan(sh, dh, th, xs, qH, 0, s, 16, ep)
                plsc.subcore_barrier()
                drain(agg, qH, s * 6256, s * 6256, 6256)
                plsc.subcore_barrier()

        qO = 2 * c + half
        zero_rows()
        plsc.subcore_barrier()
        orels = ((sCO, dCO, tCO, xC, aggCO, epad['CO']),
                 (sHO, dHO, tHO, xH, aggHO, epad['HO']),
                 (sOO, dOO, tOO, xO, aggOO, epad['OO']))
        for i, (sh, dh, th, xs, agg, ep) in enumerate(orels):
            pass_scan(sh, dh, th, xs, qO,
                      i * 2 * NP['O'] + half * NP['O'], rank8, 8, ep)
        plsc.subcore_barrier()
        for i, (sh, dh, th, xs, agg, ep) in enumerate(orels):
            drain(agg, qO, i * 2 * NP['O'] + half * NP['O'] + rank8 * 1256,
                  rank8 * 1256, 1256)

    return k


def kernel(x_C, x_H, x_O, ei_CC, ea_CC, ei_CH, ea_CH, ei_HH, ea_HH, ei_CO,
           ea_CO, ei_HO, ea_HO, ei_OO, ea_OO, cW1, cb1, cW2, cb2, hW1, hb1,
           hW2, hb2, oW1, ob1, oW2, ob2, eW1, eb1, eW2, eb2, nW1, nb1, nW2,
           nb2, linW, linb, eps, outcW, outcb, outhW, outhb):
    blk = 2000
    ei = {r: e for r, e in zip(RELS, (ei_CC, ei_CH, ei_HH, ei_CO, ei_HO,
                                      ei_OO))}
    ea = {r: e for r, e in zip(RELS, (ea_CC, ea_CH, ea_HH, ea_CO, ea_HO,
                                      ea_OO))}

    epad, src, dst, eap = {}, {}, {}, {}
    for r in RELS:
        e = ei[r].shape[1]
        ep = _ceil_to(e, EPAD_UNIT)
        epad[r] = ep
        pad = ep - e
        src[r] = jnp.concatenate([ei[r][0], jnp.zeros((pad,), jnp.int32)])
        dst[r] = jnp.concatenate([ei[r][1], jnp.zeros((pad,), jnp.int32)])
        eap[r] = jnp.concatenate(
            [ea[r], jnp.zeros((pad, ea[r].shape[1]), jnp.float32)])

    wc = jnp.einsum('dk,jkh->jdh', eW2, linW)
    bc = jnp.einsum('k,jkh->jh', eb2, linW) + linb

    x = {
        'C': _mlp2(x_C, cW1, cb1, cW2, cb2, blk),
        'H': _mlp2(x_H, hW1, hb1, hW2, hb2, blk),
        'O': _mlp2(x_O, oW1, ob1, oW2, ob2, blk),
    }
    t = {}
    for i, r in enumerate(RELS):
        t0, t1 = _edge_t(eap[r], eW1, eb1, wc[i], bc[i], wc[i + 6], bc[i + 6],
                         2048, ei[r].shape[1])
        t[r] = (t0.reshape(-1, 16), t1.reshape(-1, 16))

    sc_layer = _sc_layer_kernel(epad)

    for layer in range(2):
        xq = {p: x[p].reshape(-1, 16) for p in 'CHO'}
        aggs = sc_layer(
            xq['C'], xq['H'], xq['O'],
            src['CC'], dst['CC'], t['CC'][layer],
            src['CH'], dst['CH'], t['CH'][layer],
            src['HH'], dst['HH'], t['HH'][layer],
            src['CO'], dst['CO'], t['CO'][layer],
            src['HO'], dst['HO'], t['HO'][layer],
            src['OO'], dst['OO'], t['OO'][layer])
        agg = {r: jnp.transpose(a, (1, 0, 2))[:NT[r[1]]].reshape(-1, HID)
               for r, a in zip(RELS, aggs)}
        s = 1.0 + eps[layer * 6:layer * 6 + 6]
        x = {
            'C': _node_update(x['C'], [agg['CC']], s[0:1], nW1, nb1, nW2,
                              nb2, blk),
            'H': _node_update(x['H'], [agg['CH'], agg['HH']], s[1:3], nW1,
                              nb1, nW2, nb2, blk),
            'O': _node_update(x['O'], [agg['CO'], agg['HO'], agg['OO']],
                              s[3:6], nW1, nb1, nW2, nb2, blk),
        }
    out_c = _head(x['C'], outcW, outcb, blk)
    out_h = _head(x['H'], outhW, outhb, blk)
    return out_c, out_h


# speedup vs baseline: 1.3335x; 1.1606x over previous
import functools

import jax
import jax.numpy as jnp
from jax import lax
from jax.experimental import pallas as pl
from jax.experimental.pallas import tpu as pltpu
from jax.experimental.pallas import tpu_sc as plsc

HID = 64
RELS = ['CC', 'CH', 'HH', 'CO', 'HO', 'OO']
NT = {'C': 50000, 'H': 100000, 'O': 10000}
NP = {'C': 50048, 'H': 100096, 'O': 10048}
ACC_ROWS = 100096
EPAD_UNIT = 8192


def _ceil_to(x, m):
    return (x + m - 1) // m * m


def _mlp2_body(x_ref, w1_ref, b1_ref, w2_ref, b2_ref, o_ref, *, relu_out):
    h = jnp.maximum(x_ref[...] @ w1_ref[...] + b1_ref[...], 0.0)
    o = h @ w2_ref[...] + b2_ref[...]
    if relu_out:
        o = jnp.maximum(o, 0.0)
    o_ref[...] = o


def _mlp2(x, w1, b1, w2, b2, blk, relu_out=False):
    n, din = x.shape
    return pl.pallas_call(
        functools.partial(_mlp2_body, relu_out=relu_out),
        grid=(n // blk,),
        in_specs=[
            pl.BlockSpec((blk, din), lambda i: (i, 0)),
            pl.BlockSpec((din, HID), lambda i: (0, 0)),
            pl.BlockSpec((1, HID), lambda i: (0, 0)),
            pl.BlockSpec((HID, HID), lambda i: (0, 0)),
            pl.BlockSpec((1, HID), lambda i: (0, 0)),
        ],
        out_specs=pl.BlockSpec((blk, HID), lambda i: (i, 0)),
        out_shape=jax.ShapeDtypeStruct((n, HID), jnp.float32),
    )(x, w1, b1.reshape(1, HID), w2, b2.reshape(1, HID))


def _edge_t_body(ea_ref, w1_ref, b1_ref, w2_ref, b2_ref, wa_ref, ba_ref,
                 wb_ref, bb_ref, ta_ref, tb_ref, *, blk, n_real):
    h = jnp.maximum(ea_ref[...] @ w1_ref[...] + b1_ref[...], 0.0)
    enc = h @ w2_ref[...] + b2_ref[...]
    ta = enc @ wa_ref[...] + ba_ref[...]
    tb = enc @ wb_ref[...] + bb_ref[...]
    rows = pl.program_id(0) * blk + lax.broadcasted_iota(jnp.int32, (blk, 1), 0)
    pad = rows >= n_real
    ta_ref[...] = jnp.where(pad, -1e9, ta)
    tb_ref[...] = jnp.where(pad, -1e9, tb)


def _edge_t(ea, ew1, eb1, ew2, eb2, wa, ba, wb, bb, blk, n_real):
    n, din = ea.shape
    out = jax.ShapeDtypeStruct((n, HID), jnp.float32)
    return pl.pallas_call(
        functools.partial(_edge_t_body, blk=blk, n_real=n_real),
        grid=(n // blk,),
        in_specs=[
            pl.BlockSpec((blk, din), lambda i: (i, 0)),
            pl.BlockSpec((din, HID), lambda i: (0, 0)),
            pl.BlockSpec((1, HID), lambda i: (0, 0)),
            pl.BlockSpec((HID, HID), lambda i: (0, 0)),
            pl.BlockSpec((1, HID), lambda i: (0, 0)),
            pl.BlockSpec((HID, HID), lambda i: (0, 0)),
            pl.BlockSpec((1, HID), lambda i: (0, 0)),
            pl.BlockSpec((HID, HID), lambda i: (0, 0)),
            pl.BlockSpec((1, HID), lambda i: (0, 0)),
        ],
        out_specs=[pl.BlockSpec((blk, HID), lambda i: (i, 0))] * 2,
        out_shape=[out, out],
    )(ea, ew1, eb1.reshape(1, HID), ew2, eb2.reshape(1, HID), wa,
      ba.reshape(1, HID), wb, bb.reshape(1, HID))


def _node_update_body(*refs, k):
    x_ref = refs[0]
    agg_refs = refs[1:1 + k]
    s_ref, w1_ref, b1_ref, w2_ref, b2_ref, o_ref = refs[1 + k:]
    acc = None
    for i in range(k):
        a4 = agg_refs[i][...]
        agg = jnp.concatenate([a4[0], a4[1], a4[2], a4[3]], axis=1)
        z = x_ref[...] * s_ref[0, i] + agg
        h = jnp.maximum(z @ w1_ref[...] + b1_ref[...], 0.0)
        y = h @ w2_ref[...] + b2_ref[...]
        acc = y if acc is None else acc + y
    o_ref[...] = jnp.maximum(acc, 0.0)


def _node_update(x, aggs, scales, nw1, nb1, nw2, nb2, blk):
    n = x.shape[0]
    k = len(aggs)
    in_specs = [pl.BlockSpec((blk, HID), lambda i: (i, 0))]
    in_specs += [pl.BlockSpec((4, blk, 16), lambda i: (0, i, 0))] * k
    in_specs += [
        pl.BlockSpec((1, k), lambda i: (0, 0)),
        pl.BlockSpec((HID, HID), lambda i: (0, 0)),
        pl.BlockSpec((1, HID), lambda i: (0, 0)),
        pl.BlockSpec((HID, HID), lambda i: (0, 0)),
        pl.BlockSpec((1, HID), lambda i: (0, 0)),
    ]
    return pl.pallas_call(
        functools.partial(_node_update_body, k=k),
        grid=(n // blk,),
        in_specs=in_specs,
        out_specs=pl.BlockSpec((blk, HID), lambda i: (i, 0)),
        out_shape=jax.ShapeDtypeStruct((n, HID), jnp.float32),
    )(x, *aggs, scales.reshape(1, k), nw1, nb1.reshape(1, HID), nw2,
      nb2.reshape(1, HID))


def _head_body(x_ref, w_ref, b_ref, o_ref):
    o_ref[...] = x_ref[...] @ w_ref[...] + b_ref[...]


def _head(x, w, b, blk):
    n = x.shape[0]
    return pl.pallas_call(
        _head_body,
        grid=(n // blk,),
        in_specs=[
            pl.BlockSpec((blk, HID), lambda i: (i, 0)),
            pl.BlockSpec((HID, 1), lambda i: (0, 0)),
            pl.BlockSpec((1, 1), lambda i: (0, 0)),
        ],
        out_specs=pl.BlockSpec((blk, 1), lambda i: (i, 0)),
        out_shape=jax.ShapeDtypeStruct((n, 1), jnp.float32),
    )(x, w, b.reshape(1, 1))


BLK_E = 512


def _sc_layer_kernel(epad):
    mesh = plsc.VectorSubcoreMesh(core_axis_name="c", subcore_axis_name="s")
    f32 = jnp.float32
    i32 = jnp.int32
    out_type = [
        jax.ShapeDtypeStruct((4, NP['C'], 16), f32),
        jax.ShapeDtypeStruct((4, NP['H'], 16), f32),
        jax.ShapeDtypeStruct((4, NP['H'], 16), f32),
        jax.ShapeDtypeStruct((4, NP['O'], 16), f32),
        jax.ShapeDtypeStruct((4, NP['O'], 16), f32),
        jax.ShapeDtypeStruct((4, NP['O'], 16), f32),
    ]
    scratch_types = [
        pltpu.VMEM_SHARED((ACC_ROWS, 16), f32),
        pltpu.VMEM((BLK_E, 16), f32),
        pltpu.VMEM((BLK_E, 16), f32),
        pltpu.VMEM((BLK_E,), i32),
        pltpu.VMEM((BLK_E,), i32),
        pltpu.VMEM((4, 128), i32),
        pltpu.VMEM((4, 128), i32),
        pltpu.VMEM((4, 128), i32),
        pltpu.VMEM((391, 16), f32),
        pltpu.SemaphoreType.DMA,
    ]

    @functools.partial(pl.kernel, out_type=out_type, mesh=mesh,
                       scratch_types=scratch_types, name="gine_sc_layer",
                       compiler_params=pltpu.CompilerParams(
                           use_tc_tiling_on_sc=False))
    def k(xC, xH, xO,
          sCC, dCC, tCC, sCH, dCH, tCH, sHH, dHH, tHH,
          sCO, dCO, tCO, sHO, dHO, tHO, sOO, dOO, tOO,
          aggCC, aggCH, aggHH, aggCO, aggHO, aggOO,
          acc, tb, msg, sraw, draw, gidx, didx, tidx, zb, sem):
        c = lax.axis_index("c")
        s = lax.axis_index("s")
        half = s // 8
        rank8 = s % 8
        tio = lax.iota(i32, 16) * 4

        def zfill(i, _):
            zb[i] = jnp.zeros((16,), f32)
            return 0
        lax.fori_loop(0, 391, zfill, 0)

        def pass_scan(src_h, dst_h, t_h, x_h, q, accbase, rank, ntiles, ep):
            cnt = ep // ntiles
            nblk = cnt // BLK_E
            base = rank * cnt

            def blk_body(j, _):
                eoff = base + j * BLK_E
                pltpu.sync_copy(src_h.at[pl.ds(eoff, BLK_E)], sraw)
                pltpu.sync_copy(dst_h.at[pl.ds(eoff, BLK_E)], draw)
                for u in range(4):
                    for l in range(8):
                        o = u * 128 + l * 16
                        sv = sraw[pl.ds(o, 16)]
                        gidx[u, pl.ds(l * 16, 16)] = (sv << 2) + q
                        dv = draw[pl.ds(o, 16)]
                        didx[u, pl.ds(l * 16, 16)] = dv + accbase
                        tidx[u, pl.ds(l * 16, 16)] = ((eoff + o) << 2) + q + tio
                cps = []
                for u in range(4):
                    cps.append(pltpu.async_copy(
                        t_h.at[tidx.at[u]], tb.at[pl.ds(u * 128, 128)], sem))
                    cps.append(pltpu.async_copy(
                        x_h.at[gidx.at[u]], msg.at[pl.ds(u * 128, 128)], sem))
                for cp in cps:
                    cp.wait()

                def rb(i, _):
                    b2 = i * 8
                    for k2 in range(8):
                        r2 = b2 + k2
                        msg[r2] = jnp.maximum(msg[r2] + tb[r2], 0.0)
                    return 0
                lax.fori_loop(0, BLK_E // 8, rb, 0)

                for u in range(4):
                    pltpu.sync_copy(msg.at[pl.ds(u * 128, 128)],
                                    acc.at[didx.at[u]], add=True)
                return 0

            lax.fori_loop(0, nblk, blk_body, 0)

        def drain(agg, q, a0, r0, nr):
            pltpu.sync_copy(acc.at[pl.ds(a0, nr)], agg.at[q, pl.ds(r0, nr)])

        def zero_rows():
            z0 = s * 6256
            def zbody(i, _):
                pltpu.sync_copy(zb, acc.at[pl.ds(z0 + i * 391, 391)])
                return 0
            lax.fori_loop(0, 16, zbody, 0)

        qC = 2 * c + half
        zero_rows()
        plsc.subcore_barrier()
        pass_scan(sCC, dCC, tCC, xC, qC, half * NP['C'], rank8, 8, epad['CC'])
        plsc.subcore_barrier()
        drain(aggCC, qC, half * NP['C'] + rank8 * 6256, rank8 * 6256, 6256)
        plsc.subcore_barrier()

        for (sh, dh, th, xs, agg, ep) in (
                (sCH, dCH, tCH, xC, aggCH, epad['CH']),
                (sHH, dHH, tHH, xH, aggHH, epad['HH'])):
            for k2 in range(2):
                qH = 2 * c + k2
                zero_rows()
                plsc.subcore_barrier()
                pass_scan(sh, dh, th, xs, qH, 0, s, 16, ep)
                plsc.subcore_barrier()
                drain(agg, qH, s * 6256, s * 6256, 6256)
                plsc.subcore_barrier()

        qO = 2 * c + half
        zero_rows()
        plsc.subcore_barrier()
        orels = ((sCO, dCO, tCO, xC, aggCO, epad['CO']),
                 (sHO, dHO, tHO, xH, aggHO, epad['HO']),
                 (sOO, dOO, tOO, xO, aggOO, epad['OO']))
        for i, (sh, dh, th, xs, agg, ep) in enumerate(orels):
            pass_scan(sh, dh, th, xs, qO,
                      i * 2 * NP['O'] + half * NP['O'], rank8, 8, ep)
        plsc.subcore_barrier()
        for i, (sh, dh, th, xs, agg, ep) in enumerate(orels):
            drain(agg, qO, i * 2 * NP['O'] + half * NP['O'] + rank8 * 1256,
                  rank8 * 1256, 1256)

    return k


def kernel(x_C, x_H, x_O, ei_CC, ea_CC, ei_CH, ea_CH, ei_HH, ea_HH, ei_CO,
           ea_CO, ei_HO, ea_HO, ei_OO, ea_OO, cW1, cb1, cW2, cb2, hW1, hb1,
           hW2, hb2, oW1, ob1, oW2, ob2, eW1, eb1, eW2, eb2, nW1, nb1, nW2,
           nb2, linW, linb, eps, outcW, outcb, outhW, outhb):
    blk = 2000
    ei = {r: e for r, e in zip(RELS, (ei_CC, ei_CH, ei_HH, ei_CO, ei_HO,
                                      ei_OO))}
    ea = {r: e for r, e in zip(RELS, (ea_CC, ea_CH, ea_HH, ea_CO, ea_HO,
                                      ea_OO))}

    epad, src, dst, eap = {}, {}, {}, {}
    for r in RELS:
        e = ei[r].shape[1]
        ep = _ceil_to(e, EPAD_UNIT)
        epad[r] = ep
        pad = ep - e
        src[r] = jnp.concatenate([ei[r][0], jnp.zeros((pad,), jnp.int32)])
        dst[r] = jnp.concatenate([ei[r][1], jnp.zeros((pad,), jnp.int32)])
        eap[r] = jnp.concatenate(
            [ea[r], jnp.zeros((pad, ea[r].shape[1]), jnp.float32)])

    x = {
        'C': _mlp2(x_C, cW1, cb1, cW2, cb2, blk),
        'H': _mlp2(x_H, hW1, hb1, hW2, hb2, blk),
        'O': _mlp2(x_O, oW1, ob1, oW2, ob2, blk),
    }
    t = {}
    for i, r in enumerate(RELS):
        t0, t1 = _edge_t(eap[r], eW1, eb1, eW2, eb2, linW[i], linb[i],
                         linW[i + 6], linb[i + 6], 2048, ei[r].shape[1])
        t[r] = (t0.reshape(-1, 16), t1.reshape(-1, 16))

    sc_layer = _sc_layer_kernel(epad)

    for layer in range(2):
        xq = {p: x[p].reshape(-1, 16) for p in 'CHO'}
        aggs = sc_layer(
            xq['C'], xq['H'], xq['O'],
            src['CC'], dst['CC'], t['CC'][layer],
            src['CH'], dst['CH'], t['CH'][layer],
            src['HH'], dst['HH'], t['HH'][layer],
            src['CO'], dst['CO'], t['CO'][layer],
            src['HO'], dst['HO'], t['HO'][layer],
            src['OO'], dst['OO'], t['OO'][layer])
        agg = {r: a for r, a in zip(RELS, aggs)}
        s = 1.0 + eps[layer * 6:layer * 6 + 6]
        x = {
            'C': _node_update(x['C'], [agg['CC']], s[0:1], nW1, nb1, nW2,
                              nb2, blk),
            'H': _node_update(x['H'], [agg['CH'], agg['HH']], s[1:3], nW1,
                              nb1, nW2, nb2, blk),
            'O': _node_update(x['O'], [agg['CO'], agg['HO'], agg['OO']],
                              s[3:6], nW1, nb1, nW2, nb2, blk),
        }
    out_c = _head(x['C'], outcW, outcb, blk)
    out_h = _head(x['H'], outhW, outhb, blk)
    return out_c, out_h


# speedup vs baseline: 1.4348x; 1.0759x over previous
import functools

import jax
import jax.numpy as jnp
from jax import lax
from jax.experimental import pallas as pl
from jax.experimental.pallas import tpu as pltpu
from jax.experimental.pallas import tpu_sc as plsc

HID = 64
RELS = ['CC', 'CH', 'HH', 'CO', 'HO', 'OO']
NT = {'C': 50000, 'H': 100000, 'O': 10000}
NP = {'C': 50048, 'H': 100096, 'O': 10048}
ACC_ROWS = 100096
EPAD_UNIT = 8192


def _ceil_to(x, m):
    return (x + m - 1) // m * m


def _mlp2_body(x_ref, w1_ref, b1_ref, w2_ref, b2_ref, o_ref, *, relu_out):
    h = jnp.maximum(x_ref[...] @ w1_ref[...] + b1_ref[...], 0.0)
    o = h @ w2_ref[...] + b2_ref[...]
    if relu_out:
        o = jnp.maximum(o, 0.0)
    o_ref[...] = o


def _mlp2(x, w1, b1, w2, b2, blk, relu_out=False):
    n, din = x.shape
    return pl.pallas_call(
        functools.partial(_mlp2_body, relu_out=relu_out),
        grid=(n // blk,),
        in_specs=[
            pl.BlockSpec((blk, din), lambda i: (i, 0)),
            pl.BlockSpec((din, HID), lambda i: (0, 0)),
            pl.BlockSpec((1, HID), lambda i: (0, 0)),
            pl.BlockSpec((HID, HID), lambda i: (0, 0)),
            pl.BlockSpec((1, HID), lambda i: (0, 0)),
        ],
        out_specs=pl.BlockSpec((blk, HID), lambda i: (i, 0)),
        out_shape=jax.ShapeDtypeStruct((n, HID), jnp.float32),
    )(x, w1, b1.reshape(1, HID), w2, b2.reshape(1, HID))


def _edge_t_body(ea_ref, w1_ref, b1_ref, w2_ref, b2_ref, wa_ref, ba_ref,
                 wb_ref, bb_ref, ta_ref, tb_ref, *, blk, n_real):
    h = jnp.maximum(ea_ref[...] @ w1_ref[...] + b1_ref[...], 0.0)
    enc = h @ w2_ref[...] + b2_ref[...]
    ta = enc @ wa_ref[...] + ba_ref[...]
    tb = enc @ wb_ref[...] + bb_ref[...]
    rows = pl.program_id(0) * blk + lax.broadcasted_iota(jnp.int32, (blk, 1), 0)
    pad = rows >= n_real
    ta_ref[...] = jnp.where(pad, -1e9, ta)
    tb_ref[...] = jnp.where(pad, -1e9, tb)


def _edge_t(ea, ew1, eb1, ew2, eb2, wa, ba, wb, bb, blk, n_real):
    n, din = ea.shape
    out = jax.ShapeDtypeStruct((n, HID), jnp.float32)
    return pl.pallas_call(
        functools.partial(_edge_t_body, blk=blk, n_real=n_real),
        grid=(n // blk,),
        in_specs=[
            pl.BlockSpec((blk, din), lambda i: (i, 0)),
            pl.BlockSpec((din, HID), lambda i: (0, 0)),
            pl.BlockSpec((1, HID), lambda i: (0, 0)),
            pl.BlockSpec((HID, HID), lambda i: (0, 0)),
            pl.BlockSpec((1, HID), lambda i: (0, 0)),
            pl.BlockSpec((HID, HID), lambda i: (0, 0)),
            pl.BlockSpec((1, HID), lambda i: (0, 0)),
            pl.BlockSpec((HID, HID), lambda i: (0, 0)),
            pl.BlockSpec((1, HID), lambda i: (0, 0)),
        ],
        out_specs=[pl.BlockSpec((blk, HID), lambda i: (i, 0))] * 2,
        out_shape=[out, out],
    )(ea, ew1, eb1.reshape(1, HID), ew2, eb2.reshape(1, HID), wa,
      ba.reshape(1, HID), wb, bb.reshape(1, HID))


def _node_update_body(*refs, k):
    x_ref = refs[0]
    agg_refs = refs[1:1 + k]
    s_ref, w1_ref, b1_ref, w2_ref, b2_ref, o_ref = refs[1 + k:]
    acc = None
    for i in range(k):
        a4 = agg_refs[i][...]
        agg = jnp.concatenate([a4[0], a4[1], a4[2], a4[3]], axis=1)
        z = x_ref[...] * s_ref[0, i] + agg
        h = jnp.maximum(z @ w1_ref[...] + b1_ref[...], 0.0)
        y = h @ w2_ref[...] + b2_ref[...]
        acc = y if acc is None else acc + y
    o_ref[...] = jnp.maximum(acc, 0.0)


def _node_update(x, aggs, scales, nw1, nb1, nw2, nb2, blk):
    n = x.shape[0]
    k = len(aggs)
    in_specs = [pl.BlockSpec((blk, HID), lambda i: (i, 0))]
    in_specs += [pl.BlockSpec((4, blk, 16), lambda i: (0, i, 0))] * k
    in_specs += [
        pl.BlockSpec((1, k), lambda i: (0, 0)),
        pl.BlockSpec((HID, HID), lambda i: (0, 0)),
        pl.BlockSpec((1, HID), lambda i: (0, 0)),
        pl.BlockSpec((HID, HID), lambda i: (0, 0)),
        pl.BlockSpec((1, HID), lambda i: (0, 0)),
    ]
    return pl.pallas_call(
        functools.partial(_node_update_body, k=k),
        grid=(n // blk,),
        in_specs=in_specs,
        out_specs=pl.BlockSpec((blk, HID), lambda i: (i, 0)),
        out_shape=jax.ShapeDtypeStruct((n, HID), jnp.float32),
    )(x, *aggs, scales.reshape(1, k), nw1, nb1.reshape(1, HID), nw2,
      nb2.reshape(1, HID))


def _head_body(x_ref, w_ref, b_ref, o_ref):
    o_ref[...] = x_ref[...] @ w_ref[...] + b_ref[...]


def _head(x, w, b, blk):
    n = x.shape[0]
    return pl.pallas_call(
        _head_body,
        grid=(n // blk,),
        in_specs=[
            pl.BlockSpec((blk, HID), lambda i: (i, 0)),
            pl.BlockSpec((HID, 1), lambda i: (0, 0)),
            pl.BlockSpec((1, 1), lambda i: (0, 0)),
        ],
        out_specs=pl.BlockSpec((blk, 1), lambda i: (i, 0)),
        out_shape=jax.ShapeDtypeStruct((n, 1), jnp.float32),
    )(x, w, b.reshape(1, 1))


BLK_E = 256


def _sc_layer_kernel(epad):
    mesh = plsc.VectorSubcoreMesh(core_axis_name="c", subcore_axis_name="s")
    f32 = jnp.float32
    i32 = jnp.int32
    out_type = [
        jax.ShapeDtypeStruct((4, NP['C'], 16), f32),
        jax.ShapeDtypeStruct((4, NP['H'], 16), f32),
        jax.ShapeDtypeStruct((4, NP['H'], 16), f32),
        jax.ShapeDtypeStruct((4, NP['O'], 16), f32),
        jax.ShapeDtypeStruct((4, NP['O'], 16), f32),
        jax.ShapeDtypeStruct((4, NP['O'], 16), f32),
    ]
    scratch_types = [
        pltpu.VMEM_SHARED((ACC_ROWS, 16), f32),
        pltpu.VMEM((2, BLK_E, 16), f32),
        pltpu.VMEM((2, BLK_E, 16), f32),
        pltpu.VMEM((2, BLK_E), i32),
        pltpu.VMEM((2, BLK_E), i32),
        pltpu.VMEM((4, 128), i32),
        pltpu.VMEM((4, 128), i32),
        pltpu.VMEM((4, 128), i32),
        pltpu.VMEM((391, 16), f32),
        pltpu.SemaphoreType.DMA,
        pltpu.SemaphoreType.DMA,
        pltpu.SemaphoreType.DMA,
    ]

    @functools.partial(pl.kernel, out_type=out_type, mesh=mesh,
                       scratch_types=scratch_types, name="gine_sc_layer",
                       compiler_params=pltpu.CompilerParams(
                           use_tc_tiling_on_sc=False))
    def k(xC, xH, xO,
          sCC, dCC, tCC, sCH, dCH, tCH, sHH, dHH, tHH,
          sCO, dCO, tCO, sHO, dHO, tHO, sOO, dOO, tOO,
          aggCC, aggCH, aggHH, aggCO, aggHO, aggOO,
          acc, tb, msg, sraw, draw, gidx, didx, tidx, zb,
          sem_l, sem_g, sem_s):
        c = lax.axis_index("c")
        s = lax.axis_index("s")
        half = s // 8
        rank8 = s % 8
        tio = lax.iota(i32, 16) * 4

        def zfill(i, _):
            zb[i] = jnp.zeros((16,), f32)
            return 0
        lax.fori_loop(0, 391, zfill, 0)

        def pass_scan(src_h, dst_h, t_h, x_h, q, accbase, rank, ntiles, ep):
            cnt = ep // ntiles
            nblk = cnt // BLK_E
            base = rank * cnt

            def fire_loads(j, b):
                eoff = base + j * BLK_E
                cps = [pltpu.make_async_copy(
                           src_h.at[pl.ds(eoff, BLK_E)], sraw.at[b], sem_l),
                       pltpu.make_async_copy(
                           dst_h.at[pl.ds(eoff, BLK_E)], draw.at[b], sem_l)]
                for cp in cps:
                    cp.start()
                return cps

            def idx_compute(j, b):
                eoff = base + j * BLK_E
                for u in range(2):
                    r = 2 * b + u
                    for l in range(8):
                        o = u * 128 + l * 16
                        sv = sraw[b, pl.ds(o, 16)]
                        gidx[r, pl.ds(l * 16, 16)] = (sv << 2) + q
                        dv = draw[b, pl.ds(o, 16)]
                        didx[r, pl.ds(l * 16, 16)] = dv + accbase
                        tidx[r, pl.ds(l * 16, 16)] = ((eoff + o) << 2) + q + tio

            def fire_gathers(b):
                cps = []
                for u in range(2):
                    r = 2 * b + u
                    cps.append(pltpu.make_async_copy(
                        t_h.at[tidx.at[r]], tb.at[b, pl.ds(u * 128, 128)],
                        sem_g))
                    cps.append(pltpu.make_async_copy(
                        x_h.at[gidx.at[r]], msg.at[b, pl.ds(u * 128, 128)],
                        sem_g))
                for cp in cps:
                    cp.start()
                return cps

            def relu_add(b):
                def rb(i, _):
                    b2 = i * 8
                    for k2 in range(8):
                        r2 = b2 + k2
                        msg[b, r2] = jnp.maximum(msg[b, r2] + tb[b, r2], 0.0)
                    return 0
                lax.fori_loop(0, BLK_E // 8, rb, 0)

            def fire_scatters(b):
                cps = []
                for u in range(2):
                    r = 2 * b + u
                    cps.append(pltpu.make_async_copy(
                        msg.at[b, pl.ds(u * 128, 128)], acc.at[didx.at[r]],
                        sem_s))
                for cp in cps:
                    cp.start(add=True)
                return cps

            def pair_body(i, _):
                j0 = 2 * i
                l0 = fire_loads(j0, 0)
                l1 = fire_loads(j0 + 1, 1)
                for cp in l0:
                    cp.wait()
                idx_compute(j0, 0)
                g0 = fire_gathers(0)
                for cp in l1:
                    cp.wait()
                idx_compute(j0 + 1, 1)
                g1 = fire_gathers(1)
                for cp in g0:
                    cp.wait()
                relu_add(0)
                s0 = fire_scatters(0)
                for cp in g1:
                    cp.wait()
                relu_add(1)
                s1 = fire_scatters(1)
                for cp in s0 + s1:
                    cp.wait()
                return 0

            lax.fori_loop(0, nblk // 2, pair_body, 0)

        def drain(agg, q, a0, r0, nr):
            pltpu.sync_copy(acc.at[pl.ds(a0, nr)], agg.at[q, pl.ds(r0, nr)])

        def zero_rows():
            z0 = s * 6256
            def zbody(i, _):
                pltpu.sync_copy(zb, acc.at[pl.ds(z0 + i * 391, 391)])
                return 0
            lax.fori_loop(0, 16, zbody, 0)

        qC = 2 * c + half
        zero_rows()
        plsc.subcore_barrier()
        pass_scan(sCC, dCC, tCC, xC, qC, half * NP['C'], rank8, 8, epad['CC'])
        plsc.subcore_barrier()
        drain(aggCC, qC, half * NP['C'] + rank8 * 6256, rank8 * 6256, 6256)
        plsc.subcore_barrier()

        for (sh, dh, th, xs, agg, ep) in (
                (sCH, dCH, tCH, xC, aggCH, epad['CH']),
                (sHH, dHH, tHH, xH, aggHH, epad['HH'])):
            for k2 in range(2):
                qH = 2 * c + k2
                zero_rows()
                plsc.subcore_barrier()
                pass_scan(sh, dh, th, xs, qH, 0, s, 16, ep)
                plsc.subcore_barrier()
                drain(agg, qH, s * 6256, s * 6256, 6256)
                plsc.subcore_barrier()

        qO = 2 * c + half
        zero_rows()
        plsc.subcore_barrier()
        orels = ((sCO, dCO, tCO, xC, aggCO, epad['CO']),
                 (sHO, dHO, tHO, xH, aggHO, epad['HO']),
                 (sOO, dOO, tOO, xO, aggOO, epad['OO']))
        for i, (sh, dh, th, xs, agg, ep) in enumerate(orels):
            pass_scan(sh, dh, th, xs, qO,
                      i * 2 * NP['O'] + half * NP['O'], rank8, 8, ep)
        plsc.subcore_barrier()
        for i, (sh, dh, th, xs, agg, ep) in enumerate(orels):
            drain(agg, qO, i * 2 * NP['O'] + half * NP['O'] + rank8 * 1256,
                  rank8 * 1256, 1256)

    return k


def kernel(x_C, x_H, x_O, ei_CC, ea_CC, ei_CH, ea_CH, ei_HH, ea_HH, ei_CO,
           ea_CO, ei_HO, ea_HO, ei_OO, ea_OO, cW1, cb1, cW2, cb2, hW1, hb1,
           hW2, hb2, oW1, ob1, oW2, ob2, eW1, eb1, eW2, eb2, nW1, nb1, nW2,
           nb2, linW, linb, eps, outcW, outcb, outhW, outhb):
    blk = 2000
    ei = {r: e for r, e in zip(RELS, (ei_CC, ei_CH, ei_HH, ei_CO, ei_HO,
                                      ei_OO))}
    ea = {r: e for r, e in zip(RELS, (ea_CC, ea_CH, ea_HH, ea_CO, ea_HO,
                                      ea_OO))}

    epad, src, dst, eap = {}, {}, {}, {}
    for r in RELS:
        e = ei[r].shape[1]
        ep = _ceil_to(e, EPAD_UNIT)
        epad[r] = ep
        pad = ep - e
        src[r] = jnp.concatenate([ei[r][0], jnp.zeros((pad,), jnp.int32)])
        dst[r] = jnp.concatenate([ei[r][1], jnp.zeros((pad,), jnp.int32)])
        eap[r] = jnp.concatenate(
            [ea[r], jnp.zeros((pad, ea[r].shape[1]), jnp.float32)])

    x = {
        'C': _mlp2(x_C, cW1, cb1, cW2, cb2, blk),
        'H': _mlp2(x_H, hW1, hb1, hW2, hb2, blk),
        'O': _mlp2(x_O, oW1, ob1, oW2, ob2, blk),
    }
    t = {}
    for i, r in enumerate(RELS):
        t0, t1 = _edge_t(eap[r], eW1, eb1, eW2, eb2, linW[i], linb[i],
                         linW[i + 6], linb[i + 6], 2048, ei[r].shape[1])
        t[r] = (t0.reshape(-1, 16), t1.reshape(-1, 16))

    sc_layer = _sc_layer_kernel(epad)

    for layer in range(2):
        xq = {p: x[p].reshape(-1, 16) for p in 'CHO'}
        aggs = sc_layer(
            xq['C'], xq['H'], xq['O'],
            src['CC'], dst['CC'], t['CC'][layer],
            src['CH'], dst['CH'], t['CH'][layer],
            src['HH'], dst['HH'], t['HH'][layer],
            src['CO'], dst['CO'], t['CO'][layer],
            src['HO'], dst['HO'], t['HO'][layer],
            src['OO'], dst['OO'], t['OO'][layer])
        agg = {r: a for r, a in zip(RELS, aggs)}
        s = 1.0 + eps[layer * 6:layer * 6 + 6]
        x = {
            'C': _node_update(x['C'], [agg['CC']], s[0:1], nW1, nb1, nW2,
                              nb2, blk),
            'H': _node_update(x['H'], [agg['CH'], agg['HH']], s[1:3], nW1,
                              nb1, nW2, nb2, blk),
            'O': _node_update(x['O'], [agg['CO'], agg['HO'], agg['OO']],
                              s[3:6], nW1, nb1, nW2, nb2, blk),
        }
    out_c = _head(x['C'], outcW, outcb, blk)
    out_h = _head(x['H'], outhW, outhb, blk)
    return out_c, out_h
